# baseline (speedup 1.0000x reference)
import functools

import jax
import jax.numpy as jnp
from jax import lax
from jax.experimental import pallas as pl
from jax.experimental.pallas import tpu as pltpu
from jax.experimental.pallas import tpu_sc as plsc

NUM_CLASSES_ = 100000
NUM_VIEWS_ = 8
BATCH_ = 16384
NPLANE = NUM_VIEWS_ * 3
QUARTER = BATCH_ // 4
ALIGNED = 99968
NTAIL = NUM_CLASSES_ - ALIGNED
_CH_START = (0, 25088, 50176, 75264)
_CH_LEN = (25088, 25088, 25088, 24704)

_info = plsc.get_sparse_core_info()
NC, NS = _info.num_cores, _info.num_subcores


@functools.partial(
    pl.kernel,
    mesh=plsc.VectorSubcoreMesh(core_axis_name="c", subcore_axis_name="s"),
    out_type=jax.ShapeDtypeStruct((3, NUM_VIEWS_, BATCH_), jnp.float32),
    scratch_types=[
        pltpu.VMEM((NUM_CLASSES_,), jnp.float32),
        pltpu.VMEM((BATCH_,), jnp.int32),
        pltpu.VMEM((QUARTER,), jnp.float32),
        pltpu.VMEM((QUARTER,), jnp.float32),
        pltpu.VMEM((128,), jnp.float32),
        pltpu.SemaphoreType.DMA,
        pltpu.SemaphoreType.DMA,
        pltpu.SemaphoreType.DMA,
        pltpu.SemaphoreType.DMA,
    ],
    compiler_params=pltpu.CompilerParams(
        use_tc_tiling_on_sc=True, needs_layout_passes=False
    ),
)
def _gather_planes(
    idx_hbm, table_hbm, tail_hbm, out_hbm,
    plane_v, idx_v, out0_v, out1_v, tail_v,
    sem_p, sem_i, sem_o0, sem_o1,
):
    wid = lax.axis_index("s") * NC + lax.axis_index("c")

    @pl.when(wid < NPLANE)
    def _():
        c = wid // NUM_VIEWS_
        v = wid % NUM_VIEWS_
        idx_cp = pltpu.async_copy(idx_hbm, idx_v, sem_i)
        tail_cp = pltpu.async_copy(tail_hbm.at[c, v], tail_v, sem_i)
        plane_cps = [
            pltpu.async_copy(
                table_hbm.at[c, v, pl.ds(s, l)],
                plane_v.at[pl.ds(s, l)],
                sem_p,
            )
            for s, l in zip(_CH_START, _CH_LEN)
        ]
        idx_cp.wait()
        tail_cp.wait()
        for cp in plane_cps:
            cp.wait()

        out_bufs = (out0_v, out1_v)
        out_sems = (sem_o0, sem_o1)
        out_cps = [None, None]
        for q in range(4):
            buf = q % 2
            ob = out_bufs[buf]

            def body(k, carry):
                ii = idx_v[pl.ds(q * QUARTER + k * 16, 16)]
                base = plsc.load_gather(plane_v, [ii])
                it = jnp.maximum(ii - ALIGNED, 0)
                tval = plsc.load_gather(tail_v, [it])
                ob[pl.ds(k * 16, 16)] = jnp.where(ii >= ALIGNED, tval, base)
                return carry

            if out_cps[buf] is not None:
                out_cps[buf].wait()
            lax.fori_loop(0, QUARTER // 16, body, 0, unroll=4)
            out_cps[buf] = pltpu.async_copy(
                ob,
                out_hbm.at[c, v, pl.ds(q * QUARTER, QUARTER)],
                out_sems[buf],
            )
        out_cps[0].wait()
        out_cps[1].wait()


def kernel(class_indices, camera_pos):
    idx = class_indices.astype(jnp.int32)
    tab = camera_pos.transpose(2, 1, 0)
    tail = jnp.pad(camera_pos[ALIGNED:], ((0, 128 - NTAIL), (0, 0), (0, 0))).transpose(
        2, 1, 0
    )
    out = _gather_planes(idx, tab, tail)
    return out.transpose(2, 1, 0)


# speedup vs baseline: 1.1123x; 1.1123x over previous
import functools

import jax
import jax.numpy as jnp
from jax import lax
from jax.experimental import pallas as pl
from jax.experimental.pallas import tpu as pltpu
from jax.experimental.pallas import tpu_sc as plsc

NUM_CLASSES_ = 100000
NUM_VIEWS_ = 8
BATCH_ = 16384
NPLANE = NUM_VIEWS_ * 3
HALF = BATCH_ // 2

_info = plsc.get_sparse_core_info()
NC, NS = _info.num_cores, _info.num_subcores


@functools.partial(
    pl.kernel,
    mesh=plsc.VectorSubcoreMesh(core_axis_name="c", subcore_axis_name="s"),
    out_type=jax.ShapeDtypeStruct((3, NUM_VIEWS_, BATCH_), jnp.float32),
    scratch_types=[
        pltpu.VMEM((NUM_CLASSES_,), jnp.float32),
        pltpu.VMEM((HALF,), jnp.int32),
        pltpu.VMEM((HALF,), jnp.float32),
    ],
    compiler_params=pltpu.CompilerParams(
        use_tc_tiling_on_sc=True, needs_layout_passes=False
    ),
)
def _gather_planes(idx_hbm, table_hbm, out_hbm, plane_v, idx_v, out_v):
    wid = lax.axis_index("s") * NC + lax.axis_index("c")

    @pl.when(wid < NPLANE)
    def _():
        c = wid // NUM_VIEWS_
        v = wid % NUM_VIEWS_
        pltpu.sync_copy(table_hbm.at[c, v], plane_v)
        for h in range(2):
            pltpu.sync_copy(idx_hbm.at[pl.ds(h * HALF, HALF)], idx_v)

            def body(k, carry):
                ii = idx_v[pl.ds(k * 16, 16)]
                out_v[pl.ds(k * 16, 16)] = plsc.load_gather(plane_v, [ii])
                return carry

            lax.fori_loop(0, HALF // 16, body, 0, unroll=1)
            pltpu.sync_copy(out_v, out_hbm.at[c, v, pl.ds(h * HALF, HALF)])


def kernel(class_indices, camera_pos):
    idx = class_indices.astype(jnp.int32)
    tab = camera_pos.transpose(2, 1, 0)
    out = _gather_planes(idx, tab)
    return out.transpose(2, 1, 0)


# speedup vs baseline: 1.1170x; 1.0043x over previous
import functools

import jax
import jax.numpy as jnp
from jax import lax
from jax.experimental import pallas as pl
from jax.experimental.pallas import tpu as pltpu
from jax.experimental.pallas import tpu_sc as plsc

NUM_CLASSES_ = 100000
NUM_VIEWS_ = 8
BATCH_ = 16384
NPLANE = NUM_VIEWS_ * 3
HALF = BATCH_ // 2

_info = plsc.get_sparse_core_info()
NC, NS = _info.num_cores, _info.num_subcores


@functools.partial(
    pl.kernel,
    mesh=plsc.VectorSubcoreMesh(core_axis_name="c", subcore_axis_name="s"),
    out_type=jax.ShapeDtypeStruct((3, NUM_VIEWS_, BATCH_), jnp.float32),
    scratch_types=[
        pltpu.VMEM((NUM_CLASSES_,), jnp.float32),
        pltpu.VMEM((HALF,), jnp.int32),
        pltpu.VMEM((HALF,), jnp.float32),
    ],
    compiler_params=pltpu.CompilerParams(
        use_tc_tiling_on_sc=True,
        needs_layout_passes=False,
        skip_device_barrier=True,
    ),
)
def _gather_planes(idx_hbm, table_hbm, out_hbm, plane_v, idx_v, out_v):
    wid = lax.axis_index("s") * NC + lax.axis_index("c")

    @pl.when(wid < NPLANE)
    def _():
        c = wid // NUM_VIEWS_
        v = wid % NUM_VIEWS_
        pltpu.sync_copy(table_hbm.at[c, v], plane_v)
        for h in range(2):
            pltpu.sync_copy(idx_hbm.at[pl.ds(h * HALF, HALF)], idx_v)

            def body(k, carry):
                ii = idx_v[pl.ds(k * 16, 16)]
                out_v[pl.ds(k * 16, 16)] = plsc.load_gather(plane_v, [ii])
                return carry

            lax.fori_loop(0, HALF // 16, body, 0, unroll=1)
            pltpu.sync_copy(out_v, out_hbm.at[c, v, pl.ds(h * HALF, HALF)])


def kernel(class_indices, camera_pos):
    idx = class_indices.astype(jnp.int32)
    tab = camera_pos.transpose(2, 1, 0)
    out = _gather_planes(idx, tab)
    return out.transpose(2, 1, 0)


# speedup vs baseline: 1.2479x; 1.1172x over previous
import functools

import jax
import jax.numpy as jnp
from jax import lax
from jax.experimental import pallas as pl
from jax.experimental.pallas import tpu as pltpu
from jax.experimental.pallas import tpu_sc as plsc

NUM_CLASSES_ = 100000
NUM_VIEWS_ = 8
BATCH_ = 16384
NPLANE = NUM_VIEWS_ * 3
QUARTER = BATCH_ // 4

_info = plsc.get_sparse_core_info()
NC, NS = _info.num_cores, _info.num_subcores


@functools.partial(
    pl.kernel,
    mesh=plsc.VectorSubcoreMesh(core_axis_name="c", subcore_axis_name="s"),
    out_type=jax.ShapeDtypeStruct((3, NUM_VIEWS_, BATCH_), jnp.float32),
    scratch_types=[
        pltpu.VMEM((NUM_CLASSES_,), jnp.float32),
        pltpu.VMEM((QUARTER,), jnp.int32),
        pltpu.VMEM((QUARTER,), jnp.int32),
        pltpu.VMEM((QUARTER,), jnp.float32),
        pltpu.VMEM((QUARTER,), jnp.float32),
        pltpu.VMEM_SHARED((BATCH_,), jnp.int32),
        pltpu.SemaphoreType.DMA,
        pltpu.SemaphoreType.DMA,
        pltpu.SemaphoreType.DMA,
        pltpu.SemaphoreType.DMA,
        pltpu.SemaphoreType.DMA,
    ],
    compiler_params=pltpu.CompilerParams(
        use_tc_tiling_on_sc=True, needs_layout_passes=False
    ),
)
def _gather_planes(
    idx_hbm, table_hbm, out_hbm,
    plane_v, idx0_v, idx1_v, out0_v, out1_v, sidx,
    sem_p, sem_i0, sem_i1, sem_o0, sem_o1,
):
    cid = lax.axis_index("c")
    sid = lax.axis_index("s")
    wid = sid * NC + cid
    c = wid // NUM_VIEWS_
    v = wid % NUM_VIEWS_

    @pl.when(wid < NPLANE)
    def _():
        pltpu.async_copy(table_hbm.at[c, v], plane_v, sem_p)

    @pl.when(sid == 0)
    def _():
        pltpu.sync_copy(idx_hbm, sidx)

    plsc.subcore_barrier()

    @pl.when(wid < NPLANE)
    def _():
        idx_bufs = (idx0_v, idx1_v)
        idx_sems = (sem_i0, sem_i1)
        out_bufs = (out0_v, out1_v)
        out_sems = (sem_o0, sem_o1)
        idx_cps = [None, None]
        out_cps = [None, None]
        for q in range(2):
            idx_cps[q] = pltpu.async_copy(
                sidx.at[pl.ds(q * QUARTER, QUARTER)], idx_bufs[q], idx_sems[q]
            )
        pltpu.make_async_copy(table_hbm.at[c, v], plane_v, sem_p).wait()
        for q in range(4):
            b = q % 2
            ib = idx_bufs[b]
            ob = out_bufs[b]
            idx_cps[b].wait()

            def body(k, carry):
                ii = ib[pl.ds(k * 16, 16)]
                ob[pl.ds(k * 16, 16)] = plsc.load_gather(plane_v, [ii])
                return carry

            if out_cps[b] is not None:
                out_cps[b].wait()
            lax.fori_loop(0, QUARTER // 16, body, 0, unroll=1)
            out_cps[b] = pltpu.async_copy(
                ob, out_hbm.at[c, v, pl.ds(q * QUARTER, QUARTER)], out_sems[b]
            )
            if q + 2 < 4:
                idx_cps[b] = pltpu.async_copy(
                    sidx.at[pl.ds((q + 2) * QUARTER, QUARTER)],
                    idx_bufs[b],
                    idx_sems[b],
                )
        out_cps[0].wait()
        out_cps[1].wait()


def kernel(class_indices, camera_pos):
    idx = class_indices.astype(jnp.int32)
    tab = camera_pos.transpose(2, 1, 0)
    out = _gather_planes(idx, tab)
    return out.transpose(2, 1, 0)
